# Initial kernel scaffold; baseline (speedup 1.0000x reference)
import jax
import jax.numpy as jnp
from jax.experimental import pallas as pl


def kernel(dense_input, sparse_input, bias, weight_dense, weight_sparse, embed_tables):
    raise NotImplementedError("write your pallas kernel here")


# speedup vs baseline: 8.4483x; 8.4483x over previous
import functools

import jax
import jax.numpy as jnp
from jax import lax
from jax.experimental import pallas as pl
from jax.experimental.pallas import tpu as pltpu
from jax.experimental.pallas import tpu_sc as plsc

B = 4096
F = 26
D_DENSE = 13
FEAT = 4000
DIM = 16
TOTAL = F * FEAT
WCOL = F * DIM
ROW = 512

NC = 2
NS = 16
NW = NC * NS
NB = B // NW
CHUNK = 4
NCHUNK = NB // CHUNK
ROWS_PER_CHUNK = CHUNK * F

BT = 1000


def _build_table_body(emb_ref, w_ref, t_ref):
    for j in range(F):
        t_ref[:, j * DIM:(j + 1) * DIM] = emb_ref[j, :, :]
    w = w_ref[0, 0, :].reshape(BT, 1)
    t_ref[:, WCOL:] = jnp.concatenate(
        [w, jnp.zeros((BT, ROW - WCOL - 1), jnp.float32)], axis=1)


def _build_table(embed_tables, weight_sparse):
    return pl.pallas_call(
        _build_table_body,
        grid=(TOTAL // BT,),
        in_specs=[
            pl.BlockSpec((F, BT, DIM), lambda t: (0, t, 0)),
            pl.BlockSpec((1, 1, BT), lambda t: (t, 0, 0)),
        ],
        out_specs=pl.BlockSpec((BT, ROW), lambda t: (t, 0)),
        out_shape=jax.ShapeDtypeStruct((TOTAL, ROW), jnp.float32),
    )(embed_tables, weight_sparse.reshape(TOTAL // BT, 1, BT))


def _sc_gather_cross(table, sp_flat):
    mesh = plsc.VectorSubcoreMesh(core_axis_name="c", subcore_axis_name="s")

    @functools.partial(
        pl.kernel,
        mesh=mesh,
        out_type=jax.ShapeDtypeStruct((B, DIM), jnp.float32),
        scratch_types=[
            pltpu.VMEM((NB * F,), jnp.int32),
            pltpu.VMEM((ROWS_PER_CHUNK, ROW), jnp.float32),
            pltpu.VMEM((NB, DIM), jnp.float32),
            pltpu.SemaphoreType.DMA,
        ],
    )
    def k(table_hbm, sp_hbm, out_hbm, idx_v, rows_v, out_v, sem):
        wid = lax.axis_index("s") * NC + lax.axis_index("c")
        base = wid * (NB * F)
        pltpu.sync_copy(sp_hbm.at[pl.ds(base, NB * F)], idx_v)

        def chunk_body(c, carry):
            pltpu.async_copy(
                table_hbm.at[idx_v.at[pl.ds(c * ROWS_PER_CHUNK,
                                            ROWS_PER_CHUNK)]],
                rows_v, sem).wait()

            def b_body(bb, carry2):
                r0 = bb * F
                acc = jnp.zeros((DIM,), jnp.float32)
                for i in range(F - 1):
                    for j in range(i + 1, F):
                        acc = acc + (rows_v[r0 + i, pl.ds(j * DIM, DIM)] *
                                     rows_v[r0 + j, pl.ds(i * DIM, DIM)])
                for i in range(F):
                    acc = acc + rows_v[r0 + i, pl.ds(WCOL, DIM)]
                out_v[c * CHUNK + bb, :] = acc
                return carry2

            lax.fori_loop(0, CHUNK, b_body, 0, unroll=False)
            return carry

        lax.fori_loop(0, NCHUNK, chunk_body, 0, unroll=False)
        pltpu.sync_copy(out_v, out_hbm.at[pl.ds(wid * NB, NB)])

    return k(table, sp_flat)


def _final_body(dense_ref, wd_ref, b_ref, part_ref, o_ref):
    lin = jnp.sum(dense_ref[...] * wd_ref[...], axis=1, keepdims=True)
    cross = jnp.sum(part_ref[...], axis=1, keepdims=True)
    o_ref[...] = jax.nn.sigmoid(lin + cross + b_ref[0, 0])


def _final(dense, wd_row, bias11, partial):
    return pl.pallas_call(
        _final_body,
        out_shape=jax.ShapeDtypeStruct((B, 1), jnp.float32),
    )(dense, wd_row, bias11, partial)


def kernel(dense_input, sparse_input, bias, weight_dense, weight_sparse,
           embed_tables):
    offs = jnp.arange(F, dtype=jnp.int32) * FEAT
    sp_flat = (sparse_input + offs[None, :]).reshape(B * F)
    table = _build_table(embed_tables, weight_sparse)
    partial = _sc_gather_cross(table, sp_flat)
    return _final(dense_input, weight_dense.reshape(1, D_DENSE),
                  bias.reshape(1, 1), partial)
